# baseline (device time: 35210 ns/iter reference)
import jax
import jax.numpy as jnp
from jax import lax
from jax.experimental import pallas as pl
from jax.experimental.pallas import tpu as pltpu

C = 8


def kernel(x):
    m, n = x.shape
    mc = m // C

    def body(x_ref, out_ref, peer_x_ref, sum_ref, sx, rx, sy, ry, lc):
        my_x = lax.axis_index("x")
        my_y = lax.axis_index("y")
        other_x = 1 - my_x
        other_y = 1 - my_y
        my_col = my_y * n

        barrier_sem = pltpu.get_barrier_semaphore()
        pl.semaphore_signal(barrier_sem, inc=1, device_id=(other_x, my_y),
                            device_id_type=pl.DeviceIdType.MESH)
        pl.semaphore_signal(barrier_sem, inc=1, device_id=(my_x, other_y),
                            device_id_type=pl.DeviceIdType.MESH)
        pl.semaphore_wait(barrier_sem, 2)

        x_rdmas = []
        for c in range(C):
            rows = pl.ds(c * mc, mc)
            r = pltpu.make_async_remote_copy(
                src_ref=x_ref.at[rows],
                dst_ref=peer_x_ref.at[rows],
                send_sem=sx.at[c],
                recv_sem=rx.at[c],
                device_id=(other_x, my_y),
                device_id_type=pl.DeviceIdType.MESH,
            )
            r.start()
            x_rdmas.append(r)

        y_rdmas = []
        local_copies = []
        for c in range(C):
            rows = pl.ds(c * mc, mc)
            x_rdmas[c].wait_recv()
            sum_ref[rows, :] = x_ref[rows, :] + peer_x_ref[rows, :]
            r = pltpu.make_async_remote_copy(
                src_ref=sum_ref.at[rows],
                dst_ref=out_ref.at[rows, pl.ds(my_col, n)],
                send_sem=sy.at[c],
                recv_sem=ry.at[c],
                device_id=(my_x, other_y),
                device_id_type=pl.DeviceIdType.MESH,
            )
            r.start()
            y_rdmas.append(r)
            cp = pltpu.make_async_copy(
                sum_ref.at[rows],
                out_ref.at[rows, pl.ds(my_col, n)],
                lc.at[c],
            )
            cp.start()
            local_copies.append(cp)

        for c in range(C):
            y_rdmas[c].wait_recv()
        for c in range(C):
            local_copies[c].wait()
            x_rdmas[c].wait_send()
            y_rdmas[c].wait_send()

    return pl.pallas_call(
        body,
        out_shape=jax.ShapeDtypeStruct((m, 2 * n), x.dtype),
        in_specs=[pl.BlockSpec(memory_space=pltpu.VMEM)],
        out_specs=pl.BlockSpec(memory_space=pl.ANY),
        scratch_shapes=[
            pltpu.VMEM((m, n), x.dtype),
            pltpu.VMEM((m, n), x.dtype),
            pltpu.SemaphoreType.DMA((C,)),
            pltpu.SemaphoreType.DMA((C,)),
            pltpu.SemaphoreType.DMA((C,)),
            pltpu.SemaphoreType.DMA((C,)),
            pltpu.SemaphoreType.DMA((C,)),
        ],
        compiler_params=pltpu.CompilerParams(collective_id=0),
    )(x)


# device time: 34211 ns/iter; 1.0292x vs baseline; 1.0292x over previous
import jax
import jax.numpy as jnp
from jax import lax
from jax.experimental import pallas as pl
from jax.experimental.pallas import tpu as pltpu

C = 16


def kernel(x):
    m, n = x.shape
    mc = m // C

    def body(x_ref, out_ref, peer_x_ref, sum_ref, sx, rx, sy, ry, lc):
        my_x = lax.axis_index("x")
        my_y = lax.axis_index("y")
        other_x = 1 - my_x
        other_y = 1 - my_y
        my_col = my_y * n

        barrier_sem = pltpu.get_barrier_semaphore()
        pl.semaphore_signal(barrier_sem, inc=1, device_id=(other_x, my_y),
                            device_id_type=pl.DeviceIdType.MESH)
        pl.semaphore_signal(barrier_sem, inc=1, device_id=(my_x, other_y),
                            device_id_type=pl.DeviceIdType.MESH)
        pl.semaphore_wait(barrier_sem, 2)

        x_rdmas = []
        for c in range(C):
            rows = pl.ds(c * mc, mc)
            r = pltpu.make_async_remote_copy(
                src_ref=x_ref.at[rows],
                dst_ref=peer_x_ref.at[rows],
                send_sem=sx.at[c],
                recv_sem=rx.at[c],
                device_id=(other_x, my_y),
                device_id_type=pl.DeviceIdType.MESH,
            )
            r.start()
            x_rdmas.append(r)

        y_rdmas = []
        local_copies = []
        for c in range(C):
            rows = pl.ds(c * mc, mc)
            x_rdmas[c].wait_recv()
            sum_ref[rows, :] = x_ref[rows, :] + peer_x_ref[rows, :]
            r = pltpu.make_async_remote_copy(
                src_ref=sum_ref.at[rows],
                dst_ref=out_ref.at[rows, pl.ds(my_col, n)],
                send_sem=sy.at[c],
                recv_sem=ry.at[c],
                device_id=(my_x, other_y),
                device_id_type=pl.DeviceIdType.MESH,
            )
            r.start()
            y_rdmas.append(r)
            cp = pltpu.make_async_copy(
                sum_ref.at[rows],
                out_ref.at[rows, pl.ds(my_col, n)],
                lc.at[c],
            )
            cp.start()
            local_copies.append(cp)

        for c in range(C):
            y_rdmas[c].wait_recv()
        for c in range(C):
            local_copies[c].wait()
            x_rdmas[c].wait_send()
            y_rdmas[c].wait_send()

    return pl.pallas_call(
        body,
        out_shape=jax.ShapeDtypeStruct((m, 2 * n), x.dtype),
        in_specs=[pl.BlockSpec(memory_space=pltpu.VMEM)],
        out_specs=pl.BlockSpec(memory_space=pl.ANY),
        scratch_shapes=[
            pltpu.VMEM((m, n), x.dtype),
            pltpu.VMEM((m, n), x.dtype),
            pltpu.SemaphoreType.DMA((C,)),
            pltpu.SemaphoreType.DMA((C,)),
            pltpu.SemaphoreType.DMA((C,)),
            pltpu.SemaphoreType.DMA((C,)),
            pltpu.SemaphoreType.DMA((C,)),
        ],
        compiler_params=pltpu.CompilerParams(collective_id=0),
    )(x)


# device time: 34180 ns/iter; 1.0301x vs baseline; 1.0009x over previous
import jax
import jax.numpy as jnp
from jax import lax
from jax.experimental import pallas as pl
from jax.experimental.pallas import tpu as pltpu

C = 16


def kernel(x):
    m, n = x.shape
    mc = m // C

    def body(x_ref, out_ref, peer_x_ref, sum_ref, sx, rx, sy, ry, lc):
        my_x = lax.axis_index("x")
        my_y = lax.axis_index("y")
        other_x = 1 - my_x
        other_y = 1 - my_y
        my_col = my_y * n

        barrier_sem = pltpu.get_barrier_semaphore()
        pl.semaphore_signal(barrier_sem, inc=1, device_id=(other_x, my_y),
                            device_id_type=pl.DeviceIdType.MESH)
        pl.semaphore_signal(barrier_sem, inc=1, device_id=(my_x, other_y),
                            device_id_type=pl.DeviceIdType.MESH)
        pl.semaphore_wait(barrier_sem, 2)

        x_rdmas = []
        for c in range(C):
            rows = pl.ds(c * mc, mc)
            r = pltpu.make_async_remote_copy(
                src_ref=x_ref.at[rows],
                dst_ref=peer_x_ref.at[rows],
                send_sem=sx.at[c],
                recv_sem=rx.at[c],
                device_id=(other_x, my_y),
                device_id_type=pl.DeviceIdType.MESH,
            )
            r.start()
            x_rdmas.append(r)

        y_rdmas = []
        local_copies = []
        for c in range(C):
            rows = pl.ds(c * mc, mc)
            x_rdmas[c].wait_recv()
            sum_ref[rows, :] = x_ref[rows, :] + peer_x_ref[rows, :]
            r = pltpu.make_async_remote_copy(
                src_ref=sum_ref.at[rows],
                dst_ref=out_ref.at[rows, pl.ds(my_col, n)],
                send_sem=sy.at[c],
                recv_sem=ry.at[c],
                device_id=(my_x, other_y),
                device_id_type=pl.DeviceIdType.MESH,
            )
            r.start()
            y_rdmas.append(r)
            cp = pltpu.make_async_copy(
                sum_ref.at[rows],
                out_ref.at[rows, pl.ds(my_col, n)],
                lc.at[c],
            )
            cp.start()
            local_copies.append(cp)

        for c in range(C):
            y_rdmas[c].wait_recv()
        for c in range(C):
            local_copies[c].wait()
            x_rdmas[c].wait_send()
            y_rdmas[c].wait_send()

    return pl.pallas_call(
        body,
        out_shape=jax.ShapeDtypeStruct((m, 2 * n), x.dtype),
        in_specs=[pl.BlockSpec(memory_space=pltpu.VMEM)],
        out_specs=pl.BlockSpec(memory_space=pltpu.MemorySpace.HBM),
        scratch_shapes=[
            pltpu.VMEM((m, n), x.dtype),
            pltpu.VMEM((m, n), x.dtype),
            pltpu.SemaphoreType.DMA((C,)),
            pltpu.SemaphoreType.DMA((C,)),
            pltpu.SemaphoreType.DMA((C,)),
            pltpu.SemaphoreType.DMA((C,)),
            pltpu.SemaphoreType.DMA((C,)),
        ],
        compiler_params=pltpu.CompilerParams(collective_id=0),
    )(x)
